# baseline (speedup 1.0000x reference)
import functools

import jax
import jax.numpy as jnp
from jax import lax
from jax.experimental import pallas as pl
from jax.experimental.pallas import tpu as pltpu
from jax.experimental.pallas import tpu_sc as plsc

N_NODES = 1_000_000
HIDDEN_D = 32
INPUT_D = 32
N_LAYERS = 2
BATCH = 16384
OUT_D = 32

NW = 32
WPW = BATCH // 128 // NW
RSH = 15
KCAP = 895
NWIN = 14
CHL = 512
NCHB = 33
EL = N_NODES // 4
ML = N_NODES // 2

_SC_MESH = plsc.VectorSubcoreMesh(core_axis_name="c", subcore_axis_name="s")
_I16 = lambda: lax.iota(jnp.int32, 16)


def _worker_id():
  return lax.axis_index("s") * 2 + lax.axis_index("c")


@functools.partial(
    pl.kernel,
    mesh=_SC_MESH,
    out_type=(
        jax.ShapeDtypeStruct((BATCH, 128), jnp.float32),
        jax.ShapeDtypeStruct((BATCH, 128), jnp.float32),
    ),
    scratch_types=[
        pltpu.VMEM((128, 128), jnp.int32),
        pltpu.VMEM((WPW, 128), jnp.int32),
        pltpu.VMEM((WPW, 128), jnp.int32),
        pltpu.VMEM((128, 128), jnp.float32),
        pltpu.VMEM((128, 128), jnp.float32),
        pltpu.SemaphoreType.DMA,
        pltpu.SemaphoreType.DMA,
    ],
)
def _sc_gather(ids_hbm, embs_hbm, mem_hbm, gxl_hbm, ghl_hbm,
               idsv, exq_v, emq_v, exl_v, eml_v, sem, sem2):
  wid = _worker_id()
  pltpu.sync_copy(ids_hbm, idsv)
  for j in range(WPW):
    for c in range(8):
      v = idsv[wid * WPW + j, pl.ds(c * 16, 16)]
      exq_v[j, pl.ds(c * 16, 16)] = v >> 2
      emq_v[j, pl.ds(c * 16, 16)] = v >> 1
  for j in range(WPW):
    b0 = pl.multiple_of((wid * WPW + j) * 128, 128)
    ca = pltpu.async_copy(embs_hbm.at[exq_v.at[j]], exl_v, sem)
    cb = pltpu.async_copy(mem_hbm.at[emq_v.at[j]], eml_v, sem2)
    ca.wait()
    cb.wait()
    pltpu.sync_copy(exl_v, gxl_hbm.at[pl.ds(b0, 128)])
    pltpu.sync_copy(eml_v, ghl_hbm.at[pl.ds(b0, 128)])


def _rnn_body(ids_ref, gxl_ref, ghl_ref, wih_ref, whh_ref, bih_ref, bhh_ref,
              wdec_ref, bdec_ref, out_ref, stl_ref):
  f32 = jnp.float32
  idc = ids_ref[...]
  gxl = gxl_ref[...]
  ghl = ghl_ref[...]
  off4 = idc & 3
  x = jnp.where(off4 == 0, gxl[:, 0:32],
                jnp.where(off4 == 1, gxl[:, 32:64],
                          jnp.where(off4 == 2, gxl[:, 64:96],
                                    gxl[:, 96:128])))
  hp = jnp.where((idc & 1) == 0, ghl[:, 0:64], ghl[:, 64:128])
  h0 = hp[:, 0:32]
  h1 = hp[:, 32:64]
  h0n = jnp.tanh(
      jnp.dot(x, wih_ref[0], preferred_element_type=f32)
      + jnp.dot(h0, whh_ref[0], preferred_element_type=f32)
      + (bih_ref[0] + bhh_ref[0])[None, :])
  h1n = jnp.tanh(
      jnp.dot(h0n, wih_ref[1], preferred_element_type=f32)
      + jnp.dot(h1, whh_ref[1], preferred_element_type=f32)
      + (bih_ref[1] + bhh_ref[1])[None, :])
  out_ref[...] = (jnp.dot(h1n, wdec_ref[...], preferred_element_type=f32)
                  + bdec_ref[...][None, :])
  stl_ref[:, 0:32] = h0n
  stl_ref[:, 32:64] = h1n


_RB = 1024

_rnn_call = pl.pallas_call(
    _rnn_body,
    grid=(BATCH // _RB,),
    in_specs=[
        pl.BlockSpec((_RB, 1), lambda i: (i, 0)),
        pl.BlockSpec((_RB, 128), lambda i: (i, 0)),
        pl.BlockSpec((_RB, 128), lambda i: (i, 0)),
        pl.BlockSpec((N_LAYERS, HIDDEN_D, HIDDEN_D), lambda i: (0, 0, 0)),
        pl.BlockSpec((N_LAYERS, HIDDEN_D, HIDDEN_D), lambda i: (0, 0, 0)),
        pl.BlockSpec((N_LAYERS, HIDDEN_D), lambda i: (0, 0)),
        pl.BlockSpec((N_LAYERS, HIDDEN_D), lambda i: (0, 0)),
        pl.BlockSpec((HIDDEN_D, OUT_D), lambda i: (0, 0)),
        pl.BlockSpec((OUT_D,), lambda i: (0,)),
    ],
    out_specs=(
        pl.BlockSpec((_RB, OUT_D), lambda i: (i, 0)),
        pl.BlockSpec((_RB, 2 * HIDDEN_D), lambda i: (i, 0)),
    ),
    out_shape=(
        jax.ShapeDtypeStruct((BATCH, OUT_D), jnp.float32),
        jax.ShapeDtypeStruct((BATCH, 2 * HIDDEN_D), jnp.float32),
    ),
)


def _copy_body(src_ref, dst_ref):
  dst_ref[...] = src_ref[...]


_copy_call = pl.pallas_call(
    _copy_body,
    grid=(125,),
    in_specs=[pl.BlockSpec((4000, 128), lambda i: (i, 0))],
    out_specs=pl.BlockSpec((4000, 128), lambda i: (i, 0)),
    out_shape=jax.ShapeDtypeStruct((ML, 128), jnp.float32),
)


def _scatter_body(ids_s, base_ref, st_ref, out_ref):
  del ids_s, base_ref
  out_ref[...] = st_ref[...]


_scatter_call = pl.pallas_call(
    _scatter_body,
    grid_spec=pltpu.PrefetchScalarGridSpec(
        num_scalar_prefetch=1,
        grid=(BATCH,),
        in_specs=[
            pl.BlockSpec(memory_space=pltpu.MemorySpace.HBM),
            pl.BlockSpec((1, 1, 2 * HIDDEN_D), lambda i, ids: (i, 0, 0)),
        ],
        out_specs=pl.BlockSpec((1, 1, 2 * HIDDEN_D),
                               lambda i, ids: (ids[i], 0, 0)),
    ),
    out_shape=jax.ShapeDtypeStruct((N_NODES, 1, 2 * HIDDEN_D), jnp.float32),
    input_output_aliases={1: 0},
)


def kernel(node_embs, memory, W_ih, W_hh, b_ih, b_hh, W_dec, b_dec, ids):
  ids2 = ids.reshape(128, 128)
  ids_col = ids.reshape(BATCH, 1)
  embs_l = node_embs.reshape(EL, 128)
  mem_l = memory.reshape(ML, 128)
  gxl, ghl = _sc_gather(ids2, embs_l, mem_l)
  out, st = _rnn_call(ids_col, gxl, ghl, W_ih, W_hh, b_ih, b_hh,
                      W_dec, b_dec)
  base = _copy_call(mem_l).reshape(N_NODES, 1, 2 * HIDDEN_D)
  new_mem = _scatter_call(ids, base, st.reshape(BATCH, 1, 2 * HIDDEN_D))
  new_memory = new_mem.reshape(N_NODES, N_LAYERS, HIDDEN_D)
  return out, new_memory


# speedup vs baseline: 2.2366x; 2.2366x over previous
import functools

import jax
import jax.numpy as jnp
from jax import lax
from jax.experimental import pallas as pl
from jax.experimental.pallas import tpu as pltpu
from jax.experimental.pallas import tpu_sc as plsc

N_NODES = 1_000_000
HIDDEN_D = 32
INPUT_D = 32
N_LAYERS = 2
BATCH = 16384
OUT_D = 32

NW = 32
WPW = BATCH // 128 // NW
RSH = 15
KCAP = 895
NWIN = 14
CHL = 512
NCHB = 33
EL = N_NODES // 4
ML = N_NODES // 2

_SC_MESH = plsc.VectorSubcoreMesh(core_axis_name="c", subcore_axis_name="s")
_I16 = lambda: lax.iota(jnp.int32, 16)


def _worker_id():
  return lax.axis_index("s") * 2 + lax.axis_index("c")


@functools.partial(
    pl.kernel,
    mesh=_SC_MESH,
    out_type=(
        jax.ShapeDtypeStruct((BATCH, 128), jnp.float32),
        jax.ShapeDtypeStruct((BATCH, 128), jnp.float32),
    ),
    scratch_types=[
        pltpu.VMEM((128, 128), jnp.int32),
        pltpu.VMEM((WPW, 128), jnp.int32),
        pltpu.VMEM((WPW, 128), jnp.int32),
        pltpu.VMEM((128, 128), jnp.float32),
        pltpu.VMEM((128, 128), jnp.float32),
        pltpu.SemaphoreType.DMA,
        pltpu.SemaphoreType.DMA,
    ],
)
def _sc_gather(ids_hbm, embs_hbm, mem_hbm, gxl_hbm, ghl_hbm,
               idsv, exq_v, emq_v, exl_v, eml_v, sem, sem2):
  wid = _worker_id()
  pltpu.sync_copy(ids_hbm, idsv)
  for j in range(WPW):
    for c in range(8):
      v = idsv[wid * WPW + j, pl.ds(c * 16, 16)]
      exq_v[j, pl.ds(c * 16, 16)] = v >> 2
      emq_v[j, pl.ds(c * 16, 16)] = v >> 1
  for j in range(WPW):
    b0 = pl.multiple_of((wid * WPW + j) * 128, 128)
    ca = pltpu.async_copy(embs_hbm.at[exq_v.at[j]], exl_v, sem)
    cb = pltpu.async_copy(mem_hbm.at[emq_v.at[j]], eml_v, sem2)
    ca.wait()
    cb.wait()
    pltpu.sync_copy(exl_v, gxl_hbm.at[pl.ds(b0, 128)])
    pltpu.sync_copy(eml_v, ghl_hbm.at[pl.ds(b0, 128)])


def _rnn_body(ids_ref, gxl_ref, ghl_ref, wih_ref, whh_ref, bih_ref, bhh_ref,
              wdec_ref, bdec_ref, out_ref, stl_ref):
  f32 = jnp.float32
  idc = ids_ref[...]
  gxl = gxl_ref[...]
  ghl = ghl_ref[...]
  off4 = idc & 3
  x = jnp.where(off4 == 0, gxl[:, 0:32],
                jnp.where(off4 == 1, gxl[:, 32:64],
                          jnp.where(off4 == 2, gxl[:, 64:96],
                                    gxl[:, 96:128])))
  hp = jnp.where((idc & 1) == 0, ghl[:, 0:64], ghl[:, 64:128])
  h0 = hp[:, 0:32]
  h1 = hp[:, 32:64]
  h0n = jnp.tanh(
      jnp.dot(x, wih_ref[0], preferred_element_type=f32)
      + jnp.dot(h0, whh_ref[0], preferred_element_type=f32)
      + (bih_ref[0] + bhh_ref[0])[None, :])
  h1n = jnp.tanh(
      jnp.dot(h0n, wih_ref[1], preferred_element_type=f32)
      + jnp.dot(h1, whh_ref[1], preferred_element_type=f32)
      + (bih_ref[1] + bhh_ref[1])[None, :])
  out_ref[...] = (jnp.dot(h1n, wdec_ref[...], preferred_element_type=f32)
                  + bdec_ref[...][None, :])
  stl_ref[:, 0:32] = h0n
  stl_ref[:, 32:64] = h1n


_RB = 1024

_rnn_call = pl.pallas_call(
    _rnn_body,
    grid=(BATCH // _RB,),
    in_specs=[
        pl.BlockSpec((_RB, 1), lambda i: (i, 0)),
        pl.BlockSpec((_RB, 128), lambda i: (i, 0)),
        pl.BlockSpec((_RB, 128), lambda i: (i, 0)),
        pl.BlockSpec((N_LAYERS, HIDDEN_D, HIDDEN_D), lambda i: (0, 0, 0)),
        pl.BlockSpec((N_LAYERS, HIDDEN_D, HIDDEN_D), lambda i: (0, 0, 0)),
        pl.BlockSpec((N_LAYERS, HIDDEN_D), lambda i: (0, 0)),
        pl.BlockSpec((N_LAYERS, HIDDEN_D), lambda i: (0, 0)),
        pl.BlockSpec((HIDDEN_D, OUT_D), lambda i: (0, 0)),
        pl.BlockSpec((OUT_D,), lambda i: (0,)),
    ],
    out_specs=(
        pl.BlockSpec((_RB, OUT_D), lambda i: (i, 0)),
        pl.BlockSpec((_RB, 2 * HIDDEN_D), lambda i: (i, 0)),
    ),
    out_shape=(
        jax.ShapeDtypeStruct((BATCH, OUT_D), jnp.float32),
        jax.ShapeDtypeStruct((BATCH, 2 * HIDDEN_D), jnp.float32),
    ),
)


def _copy_body(src_ref, dst_ref):
  dst_ref[...] = src_ref[...]


_copy_call = pl.pallas_call(
    _copy_body,
    grid=(125,),
    in_specs=[pl.BlockSpec((4000, 128), lambda i: (i, 0))],
    out_specs=pl.BlockSpec((4000, 128), lambda i: (i, 0)),
    out_shape=jax.ShapeDtypeStruct((ML, 128), jnp.float32),
)


_SB = 128
_NSEM = 8


def _scatter_body(ids_s, base_ref, st_ref, out_ref, sems):
  del base_ref
  g = pl.program_id(0)

  def issue(i, c):
    k = g * _SB + i
    row = ids_s[k]
    def mk(j):
      return pltpu.make_async_copy(
          st_ref.at[pl.ds(i, 1)], out_ref.at[pl.ds(row, 1)], sems.at[j])
    for j in range(_NSEM):
      @pl.when((i & (_NSEM - 1)) == j)
      def _():
        @pl.when(i >= _NSEM)
        def _():
          mk(j).wait()
        mk(j).start()
    return c
  lax.fori_loop(0, _SB, issue, 0)

  for j in range(_NSEM):
    pltpu.make_async_copy(
        st_ref.at[pl.ds(0, 1)], out_ref.at[pl.ds(0, 1)], sems.at[j]).wait()


_scatter_call = pl.pallas_call(
    _scatter_body,
    grid_spec=pltpu.PrefetchScalarGridSpec(
        num_scalar_prefetch=1,
        grid=(BATCH // _SB,),
        in_specs=[
            pl.BlockSpec(memory_space=pltpu.MemorySpace.HBM),
            pl.BlockSpec((_SB, 2 * HIDDEN_D), lambda i, ids: (i, 0)),
        ],
        out_specs=pl.BlockSpec(memory_space=pltpu.MemorySpace.HBM),
        scratch_shapes=[pltpu.SemaphoreType.DMA((_NSEM,))],
    ),
    out_shape=jax.ShapeDtypeStruct((N_NODES, 2 * HIDDEN_D), jnp.float32),
    input_output_aliases={1: 0},
)


def kernel(node_embs, memory, W_ih, W_hh, b_ih, b_hh, W_dec, b_dec, ids):
  ids2 = ids.reshape(128, 128)
  ids_col = ids.reshape(BATCH, 1)
  embs_l = node_embs.reshape(EL, 128)
  mem_l = memory.reshape(ML, 128)
  gxl, ghl = _sc_gather(ids2, embs_l, mem_l)
  out, st = _rnn_call(ids_col, gxl, ghl, W_ih, W_hh, b_ih, b_hh,
                      W_dec, b_dec)
  base = _copy_call(mem_l).reshape(N_NODES, 2 * HIDDEN_D)
  new_mem = _scatter_call(ids, base, st)
  new_memory = new_mem.reshape(N_NODES, N_LAYERS, HIDDEN_D)
  return out, new_memory
